# baseline (speedup 1.0000x reference)
import functools

import jax
import jax.numpy as jnp
from jax import lax
from jax.experimental import pallas as pl
from jax.experimental.pallas import tpu as pltpu
from jax.experimental.pallas import tpu_sc as plsc


def kernel(input, table):
    B0, B1 = input.shape
    V, D = table.shape

    info = plsc.get_sparse_core_info()
    NC, NS = info.num_cores, info.num_subcores
    NW = NC * NS
    n_ch = B0 // NW
    NB = 4

    idx = input.astype(jnp.int32)
    mesh = plsc.VectorSubcoreMesh(core_axis_name="c", subcore_axis_name="s")

    @functools.partial(
        pl.kernel,
        out_type=jax.ShapeDtypeStruct((B0, B1, D), jnp.float32),
        mesh=mesh,
        compiler_params=pltpu.CompilerParams(use_tc_tiling_on_sc=True),
        scratch_types=[
            pltpu.VMEM((n_ch, B1), jnp.int32),
            [pltpu.VMEM((B1, D), jnp.float32) for _ in range(NB)],
            [pltpu.SemaphoreType.DMA for _ in range(NB)],
            [pltpu.SemaphoreType.DMA for _ in range(NB)],
        ],
    )
    def gather_k(table_hbm, idx_hbm, out_hbm, idx_v, bufs, sg, so):
        wid = lax.axis_index("s") * NC + lax.axis_index("c")
        base = wid * n_ch
        pltpu.sync_copy(idx_hbm.at[pl.ds(base, n_ch)], idx_v)

        def fire_g(c, b):
            pltpu.async_copy(table_hbm.at[idx_v.at[c]], bufs[b], sg[b])

        def fire_o(c, b):
            pltpu.async_copy(bufs[b], out_hbm.at[base + c], so[b])

        def wait(sem, b):
            pltpu.make_async_copy(out_hbm.at[0], bufs[b], sem[b]).wait()

        fire_g(0, 0)
        fire_g(1, 1)

        for c in range(NB):
            wait(sg, c)
            fire_o(c, c)
            if c >= 2:
                wait(so, c - 2)
            fire_g(c + 2, (c + 2) % NB)

        @pl.loop(NB, n_ch, step=NB)
        def body(j):
            for b in range(NB):
                c = j + b
                wait(sg, b)
                fire_o(c, b)
                wait(so, (b + 2) % NB)
                fire_g(jnp.minimum(c + 2, n_ch - 1), (b + 2) % NB)

        for b in (0, 1):
            wait(sg, b)
        for b in (2, 3):
            wait(so, b)

    return gather_k(table, idx)


# speedup vs baseline: 1.9049x; 1.9049x over previous
import functools

import jax
import jax.numpy as jnp
from jax import lax
from jax.experimental import pallas as pl
from jax.experimental.pallas import tpu as pltpu
from jax.experimental.pallas import tpu_sc as plsc


def kernel(input, table):
    B0, B1 = input.shape
    V, D = table.shape

    info = plsc.get_sparse_core_info()
    NC, NS = info.num_cores, info.num_subcores
    NW = NC * NS
    W = B0 // NW
    n_ch = B1
    NB = 5

    idx_t = jnp.transpose(input.astype(jnp.int32))
    mesh = plsc.VectorSubcoreMesh(core_axis_name="c", subcore_axis_name="s")

    @functools.partial(
        pl.kernel,
        out_type=jax.ShapeDtypeStruct((B1, B0, D), jnp.float32),
        mesh=mesh,
        scratch_types=[
            pltpu.VMEM((n_ch, W), jnp.int32),
            [pltpu.VMEM((W, D), jnp.float32) for _ in range(NB)],
            [pltpu.SemaphoreType.DMA for _ in range(NB)],
            [pltpu.SemaphoreType.DMA for _ in range(NB)],
        ],
    )
    def gather_k(table_hbm, idx_hbm, out_hbm, idx_v, bufs, sg, so):
        wid = lax.axis_index("s") * NC + lax.axis_index("c")
        base = wid * W
        pltpu.sync_copy(idx_hbm.at[:, pl.ds(base, W)], idx_v)

        def fire_g(c, b):
            pltpu.async_copy(table_hbm.at[idx_v.at[c]], bufs[b], sg[b])

        def fire_o(c, b):
            pltpu.async_copy(bufs[b], out_hbm.at[c, pl.ds(base, W)], so[b])

        def wait(sem, b):
            pltpu.make_async_copy(table_hbm.at[pl.ds(0, W)], bufs[b], sem[b]).wait()

        for c in range(3):
            fire_g(c, c)

        for c in range(NB):
            wait(sg, c)
            fire_o(c, c)
            if c >= 2:
                wait(so, c - 2)
            fire_g(c + 3, (c + 3) % NB)

        @pl.loop(NB, n_ch, step=NB)
        def body(j):
            for b in range(NB):
                c = j + b
                wait(sg, b)
                fire_o(c, b)
                wait(so, (b + 3) % NB)
                fire_g(jnp.minimum(c + 3, n_ch - 1), (b + 3) % NB)

        for b in range(3):
            wait(sg, b)
        for b in (3, 4):
            wait(so, b)

    out_t = gather_k(table, idx_t)
    return jnp.transpose(out_t, (1, 0, 2))


# speedup vs baseline: 1.9268x; 1.0115x over previous
import functools

import jax
import jax.numpy as jnp
from jax import lax
from jax.experimental import pallas as pl
from jax.experimental.pallas import tpu as pltpu
from jax.experimental.pallas import tpu_sc as plsc

W = 128
NB = 5
GD = 2


def kernel(input, table):
    B0, B1 = input.shape
    V, D = table.shape

    info = plsc.get_sparse_core_info()
    NC, NS = info.num_cores, info.num_subcores
    NW = NC * NS
    RW = B0 // NW
    SUB = RW // W
    n_ch = B1 * SUB

    idx_t = jnp.transpose(input.astype(jnp.int32))
    mesh = plsc.VectorSubcoreMesh(core_axis_name="c", subcore_axis_name="s")

    @functools.partial(
        pl.kernel,
        out_type=jax.ShapeDtypeStruct((B1, B0, D), jnp.float32),
        mesh=mesh,
        scratch_types=[
            pltpu.VMEM((B1, RW), jnp.int32),
            [pltpu.VMEM((W, D), jnp.float32) for _ in range(NB)],
            [pltpu.SemaphoreType.DMA for _ in range(NB)],
            [pltpu.SemaphoreType.DMA for _ in range(NB)],
        ],
    )
    def gather_k(table_hbm, idx_hbm, out_hbm, idx_v, bufs, sg, so):
        wid = lax.axis_index("s") * NC + lax.axis_index("c")
        base = wid * RW
        pltpu.sync_copy(idx_hbm.at[:, pl.ds(base, RW)], idx_v)

        def fire_g(c, b):
            src = table_hbm.at[idx_v.at[c // SUB, pl.ds((c % SUB) * W, W)]]
            pltpu.async_copy(src, bufs[b], sg[b])

        def fire_o(c, b):
            dst = out_hbm.at[c // SUB, pl.ds(base + (c % SUB) * W, W)]
            pltpu.async_copy(bufs[b], dst, so[b])

        def wait(sem, b):
            pltpu.make_async_copy(table_hbm.at[pl.ds(0, W)], bufs[b], sem[b]).wait()

        for c in range(GD):
            fire_g(c, c)

        for c in range(NB):
            wait(sg, c)
            fire_o(c, c)
            if c >= NB - GD:
                wait(so, c - (NB - GD))
            fire_g(c + GD, (c + GD) % NB)

        @pl.loop(NB, n_ch, step=NB)
        def body(j):
            for b in range(NB):
                c = j + b
                wait(sg, b)
                fire_o(c, b)
                wait(so, (b + GD) % NB)
                fire_g(jnp.minimum(c + GD, n_ch - 1), (b + GD) % NB)

        for b in range(GD):
            wait(sg, b)
        for b in range(GD, NB):
            wait(so, b)

    out_t = gather_k(table, idx_t)
    return jnp.transpose(out_t, (1, 0, 2))
